# baseline (speedup 1.0000x reference)
import functools

import jax
import jax.numpy as jnp
from jax import lax
from jax.experimental import pallas as pl
from jax.experimental.pallas import tpu as pltpu
from jax.experimental.pallas import tpu_sc as plsc

N = 10000
E = 320000
D_IN = 128
D_OUT = 128
R = 16
B = 8

NC = 2
NS = 16
NW = NC * NS
CH = 128
EPW = E // NW
PKW = 80
EPWP = PKW * CH
EP = NW * EPWP
HPK = PKW // 2
NPAD = 10240
RPT = NPAD // NS
DUMP_ROW = NPAD - 8

NB = 400


def _transform_body(x_ref, w_ref, wc_ref, h_ref):
    w = w_ref[...].reshape(B, D_IN * D_OUT)
    wc = wc_ref[...]
    big_w = jnp.dot(wc, w, preferred_element_type=jnp.float32)
    big_w = big_w.reshape(R, D_IN, D_OUT)
    xb = x_ref[...]
    for r in range(R):
        h_ref[r] = jnp.dot(xb, big_w[r], preferred_element_type=jnp.float32)


def _combine_body(p_ref, x_ref, lw_ref, b_ref, o_ref):
    loop = jnp.dot(x_ref[...], lw_ref[...], preferred_element_type=jnp.float32)
    o_ref[...] = p_ref[0] + p_ref[1] + loop + b_ref[0]


def _sc_gather_scatter(hflat, gidx2d, dst2d, partial,
                       idx_big, dst_big, rows0, rows1, acc, sem0, sem1):
    c = lax.axis_index("c")
    s = lax.axis_index("s")
    w = s * NC + c
    row0 = pl.multiple_of(s * RPT, RPT)

    zeros16 = jnp.zeros((16,), jnp.float32)

    def zrow(i, carry):
        for j in range(D_OUT // 16):
            rows0[i, pl.ds(j * 16, 16)] = zeros16
        return carry

    lax.fori_loop(0, CH, zrow, 0)
    for t in range(RPT // CH):
        pltpu.sync_copy(rows0, acc.at[pl.ds(row0 + t * CH, CH)])
    plsc.subcore_barrier()

    cbase = pl.multiple_of(w * PKW, PKW)

    def start_gather(k, rows, sem):
        pltpu.make_async_copy(hflat.at[idx_big.at[k]], rows, sem).start()

    def wait_gather(rows, sem):
        pltpu.make_async_copy(hflat.at[idx_big.at[0]], rows, sem).wait()

    for h in range(PKW // HPK):
        pltpu.sync_copy(gidx2d.at[pl.ds(cbase + h * HPK, HPK)], idx_big)
        pltpu.sync_copy(dst2d.at[pl.ds(cbase + h * HPK, HPK)], dst_big)
        start_gather(0, rows0, sem0)

        def body(j, carry):
            k0 = 2 * j
            start_gather(k0 + 1, rows1, sem1)
            wait_gather(rows0, sem0)
            pltpu.sync_copy(rows0, acc.at[dst_big.at[k0]], add=True)

            @pl.when(j < HPK // 2 - 1)
            def _():
                start_gather(k0 + 2, rows0, sem0)

            wait_gather(rows1, sem1)
            pltpu.sync_copy(rows1, acc.at[dst_big.at[k0 + 1]], add=True)
            return carry

        lax.fori_loop(0, HPK // 2, body, 0)

    plsc.subcore_barrier()

    pltpu.sync_copy(acc.at[pl.ds(row0, RPT)], partial.at[c, pl.ds(row0, RPT)])


def kernel(x, edge_index, etypes, weight, w_comp, loop_weight, h_bias):
    src = edge_index[0]
    dst = edge_index[1]
    gidx = etypes * jnp.int32(N) + src

    h_all = pl.pallas_call(
        _transform_body,
        grid=(N // NB,),
        in_specs=[
            pl.BlockSpec((NB, D_IN), lambda i: (i, 0)),
            pl.BlockSpec((B, D_IN, D_OUT), lambda i: (0, 0, 0)),
            pl.BlockSpec((R, B), lambda i: (0, 0)),
        ],
        out_specs=pl.BlockSpec((R, NB, D_OUT), lambda i: (0, i, 0)),
        out_shape=jax.ShapeDtypeStruct((R, N, D_OUT), jnp.float32),
    )(x, weight, w_comp)
    hflat = h_all.reshape(R * N, D_OUT)

    gidx2d = jnp.pad(gidx.reshape(NW, EPW), ((0, 0), (0, EPWP - EPW))) \
        .reshape(NW * PKW, CH)
    dst2d = jnp.pad(dst.reshape(NW, EPW), ((0, 0), (0, EPWP - EPW)),
                    constant_values=DUMP_ROW).reshape(NW * PKW, CH)

    mesh = plsc.VectorSubcoreMesh(
        core_axis_name="c", subcore_axis_name="s", num_cores=NC, num_subcores=NS)
    partial = pl.kernel(
        _sc_gather_scatter,
        out_type=jax.ShapeDtypeStruct((NC, NPAD, D_OUT), jnp.float32),
        mesh=mesh,
        scratch_types=[
            pltpu.VMEM((HPK, CH), jnp.int32),
            pltpu.VMEM((HPK, CH), jnp.int32),
            pltpu.VMEM((CH, D_OUT), jnp.float32),
            pltpu.VMEM((CH, D_OUT), jnp.float32),
            pltpu.VMEM_SHARED((NPAD, D_OUT), jnp.float32),
            pltpu.SemaphoreType.DMA,
            pltpu.SemaphoreType.DMA,
        ],
    )(hflat, gidx2d, dst2d)

    bias8 = jnp.broadcast_to(h_bias, (8, D_OUT))
    out = pl.pallas_call(
        _combine_body,
        grid=(N // NB,),
        in_specs=[
            pl.BlockSpec((NC, NB, D_OUT), lambda i: (0, i, 0)),
            pl.BlockSpec((NB, D_IN), lambda i: (i, 0)),
            pl.BlockSpec((D_IN, D_OUT), lambda i: (0, 0)),
            pl.BlockSpec((8, D_OUT), lambda i: (0, 0)),
        ],
        out_specs=pl.BlockSpec((NB, D_OUT), lambda i: (i, 0)),
        out_shape=jax.ShapeDtypeStruct((N, D_OUT), jnp.float32),
    )(partial, x, loop_weight, bias8)
    return out
